# baseline (speedup 1.0000x reference)
import jax
import jax.numpy as jnp
from jax.experimental import pallas as pl
from jax.experimental.pallas import tpu as pltpu

_BATCH = 16384
_BLOCK = 512


def _mask_kernel(w1, m1, w2, m2, w3, m3, w4, m4, w5, m5,
                 o1, o2, o3, o4, o5):
    o1[...] = w1[...] * m1[...]
    o2[...] = w2[...] * m2[...]
    o3[...] = w3[...] * m3[...]
    o4[...] = w4[...] * m4[...]
    o5[...] = w5[...] * m5[...]


def _apply_masks(W1, M1, W2, M2, W3, M3, W4, M4, W5, M5):
    shapes = [jax.ShapeDtypeStruct(w.shape, jnp.float32)
              for w in (W1, W2, W3, W4, W5)]
    return pl.pallas_call(_mask_kernel, out_shape=shapes)(
        W1, M1, W2, M2, W3, M3, W4, M4, W5, M5)


def _mlp_kernel(x_ref, w1_ref, b1_ref, w2_ref, b2_ref, w3_ref, b3_ref,
                w4_ref, b4_ref, w5_ref, b5_ref,
                h1_ref, h2_ref, h3_ref, h4_ref, h5_ref):
    dn = (((1,), (1,)), ((), ()))
    bf = jnp.bfloat16

    x = x_ref[...].astype(bf)
    h1 = jax.lax.dot_general(x, w1_ref[...].astype(bf), dn,
                             preferred_element_type=jnp.float32)
    h1 = jnp.maximum(h1 + b1_ref[...], 0.0)
    h1_ref[...] = h1

    h2 = jax.lax.dot_general(h1.astype(bf), w2_ref[...].astype(bf), dn,
                             preferred_element_type=jnp.float32)
    h2 = jnp.maximum(h2 + b2_ref[...], 0.0)
    h2_ref[...] = h2

    h3 = jax.lax.dot_general(h2.astype(bf), w3_ref[...].astype(bf), dn,
                             preferred_element_type=jnp.float32)
    h3 = jnp.maximum(h3 + b3_ref[...], 0.0)
    h3_ref[...] = h3

    h4 = jax.lax.dot_general(h3.astype(bf), w4_ref[...].astype(bf), dn,
                             preferred_element_type=jnp.float32)
    h4 = h4 + b4_ref[...]
    h4_ref[...] = h4

    h5 = jax.lax.dot_general(h4.astype(bf), w5_ref[...].astype(bf), dn,
                             preferred_element_type=jnp.float32)
    h5 = h5 + b5_ref[...]
    h5_ref[...] = h5


def _mlp_kernel4(x_ref, w1_ref, b1_ref, w2_ref, b2_ref, w3_ref, b3_ref,
                 w4_ref, b4_ref,
                 h1_ref, h2_ref, h3_ref, h4_ref):
    dn = (((1,), (1,)), ((), ()))
    bf = jnp.bfloat16

    x = x_ref[...].astype(bf)
    h1 = jax.lax.dot_general(x, w1_ref[...].astype(bf), dn,
                             preferred_element_type=jnp.float32)
    h1 = jnp.maximum(h1 + b1_ref[...], 0.0)
    h1_ref[...] = h1

    h2 = jax.lax.dot_general(h1.astype(bf), w2_ref[...].astype(bf), dn,
                             preferred_element_type=jnp.float32)
    h2 = jnp.maximum(h2 + b2_ref[...], 0.0)
    h2_ref[...] = h2

    h3 = jax.lax.dot_general(h2.astype(bf), w3_ref[...].astype(bf), dn,
                             preferred_element_type=jnp.float32)
    h3 = jnp.maximum(h3 + b3_ref[...], 0.0)
    h3_ref[...] = h3

    h4 = jax.lax.dot_general(h3.astype(bf), w4_ref[...].astype(bf), dn,
                             preferred_element_type=jnp.float32)
    h4 = h4 + b4_ref[...]
    h4_ref[...] = h4


def _layer5_kernel(h4_ref, w5_ref, b5_ref, h5_ref):
    dn = (((1,), (1,)), ((), ()))
    bf = jnp.bfloat16
    h5 = jax.lax.dot_general(h4_ref[...].astype(bf), w5_ref[...].astype(bf),
                             dn, preferred_element_type=jnp.float32)
    h5_ref[...] = h5 + b5_ref[...]


def _fused_mlp(x, W1, b1, W2, b2, W3, b3, W4, b4, W5, b5, block):
    n = x.shape[0]
    d_in = x.shape[1]
    d1, d2, d3, d4, d5 = W1.shape[0], W2.shape[0], W3.shape[0], W4.shape[0], W5.shape[0]
    b1, b2, b3, b4, b5 = (b.reshape(1, -1) for b in (b1, b2, b3, b4, b5))

    def wspec(w):
        return pl.BlockSpec(w.shape, lambda i: (0, 0))

    grid = (n // block,)
    in_specs = [
        pl.BlockSpec((block, d_in), lambda i: (i, 0)),
        wspec(W1), wspec(b1),
        wspec(W2), wspec(b2),
        wspec(W3), wspec(b3),
        wspec(W4), wspec(b4),
    ]
    out_specs = [
        pl.BlockSpec((block, d1), lambda i: (i, 0)),
        pl.BlockSpec((block, d2), lambda i: (i, 0)),
        pl.BlockSpec((block, d3), lambda i: (i, 0)),
        pl.BlockSpec((block, d4), lambda i: (i, 0)),
    ]
    out_shapes = [
        jax.ShapeDtypeStruct((n, d1), jnp.float32),
        jax.ShapeDtypeStruct((n, d2), jnp.float32),
        jax.ShapeDtypeStruct((n, d3), jnp.float32),
        jax.ShapeDtypeStruct((n, d4), jnp.float32),
    ]
    h1, h2, h3, h4 = pl.pallas_call(
        _mlp_kernel4,
        grid=grid,
        in_specs=in_specs,
        out_specs=out_specs,
        out_shape=out_shapes,
        compiler_params=pltpu.CompilerParams(
            dimension_semantics=("parallel",),
        ),
    )(x, W1, b1, W2, b2, W3, b3, W4, b4)

    blk5 = 2048
    h5 = pl.pallas_call(
        _layer5_kernel,
        grid=(n // blk5,),
        in_specs=[
            pl.BlockSpec((blk5, d4), lambda i: (i, 0)),
            wspec(W5), wspec(b5),
        ],
        out_specs=pl.BlockSpec((blk5, d5), lambda i: (i, 0)),
        out_shape=jax.ShapeDtypeStruct((n, d5), jnp.float32),
        compiler_params=pltpu.CompilerParams(
            dimension_semantics=("parallel",),
        ),
    )(h4, W5, b5)
    return h1, h2, h3, h4, h5


def kernel(x, W1, b1, M1, W2, b2, M2, W3, b3, M3, W4, b4, M4, W5, b5, M5):
    Wm1, Wm2, Wm3, Wm4, Wm5 = _apply_masks(W1, M1, W2, M2, W3, M3, W4, M4,
                                           W5, M5)
    h1, h2, h3, h4, h5 = _fused_mlp(
        x, Wm1, b1, Wm2, b2, Wm3, b3, Wm4, b4, Wm5, b5, _BLOCK)
    return (h5, h1, h2, h3, h4, h5)


# speedup vs baseline: 1.0223x; 1.0223x over previous
import jax
import jax.numpy as jnp
from jax.experimental import pallas as pl
from jax.experimental.pallas import tpu as pltpu

_BLOCK = 512
_P = (1024, 896, 512, 640, 128)


def _mask_pad_kernel(w1, m1, w2, m2, w3, m3, w4, m4, w5, m5,
                     o1, o2, o3, o4, o5):
    for w, m, o in ((w1, m1, o1), (w2, m2, o2), (w3, m3, o3),
                    (w4, m4, o4), (w5, m5, o5)):
        r, c = w.shape
        o[...] = jnp.zeros(o.shape, jnp.float32)
        o[0:r, 0:c] = w[...] * m[...]


def _apply_masks_padded(W1, M1, W2, M2, W3, M3, W4, M4, W5, M5):
    p1, p2, p3, p4, p5 = _P
    d_in = W1.shape[1]
    shapes = [jax.ShapeDtypeStruct(s, jnp.float32)
              for s in ((p1, d_in), (p2, p1), (p3, p2), (p4, p3), (p5, p4))]
    return pl.pallas_call(_mask_pad_kernel, out_shape=shapes)(
        W1, M1, W2, M2, W3, M3, W4, M4, W5, M5)


def _mlp_kernel(x_ref, w1_ref, b1_ref, w2_ref, b2_ref, w3_ref, b3_ref,
                w4_ref, b4_ref, w5_ref, b5_ref,
                h1_ref, h2_ref, h3_ref, h4_ref, h5_ref):
    dn = (((1,), (1,)), ((), ()))
    bf = jnp.bfloat16
    d1, d2, d3, d4, d5 = (r.shape[1] for r in
                          (h1_ref, h2_ref, h3_ref, h4_ref, h5_ref))

    x = x_ref[...].astype(bf)
    h1 = jax.lax.dot_general(x, w1_ref[...].astype(bf), dn,
                             preferred_element_type=jnp.float32)
    h1 = jnp.maximum(h1 + b1_ref[...], 0.0)
    h1_ref[...] = h1[:, :d1]

    h2 = jax.lax.dot_general(h1.astype(bf), w2_ref[...].astype(bf), dn,
                             preferred_element_type=jnp.float32)
    h2 = jnp.maximum(h2 + b2_ref[...], 0.0)
    h2_ref[...] = h2[:, :d2]

    h3 = jax.lax.dot_general(h2.astype(bf), w3_ref[...].astype(bf), dn,
                             preferred_element_type=jnp.float32)
    h3 = jnp.maximum(h3 + b3_ref[...], 0.0)
    h3_ref[...] = h3[:, :d3]

    h4 = jax.lax.dot_general(h3.astype(bf), w4_ref[...].astype(bf), dn,
                             preferred_element_type=jnp.float32)
    h4 = h4 + b4_ref[...]
    h4_ref[...] = h4[:, :d4]

    h5 = jax.lax.dot_general(h4.astype(bf), w5_ref[...].astype(bf), dn,
                             preferred_element_type=jnp.float32)
    h5 = h5 + b5_ref[...]
    h5_ref[...] = h5[:, :d5]


def kernel(x, W1, b1, M1, W2, b2, M2, W3, b3, M3, W4, b4, M4, W5, b5, M5):
    n, d_in = x.shape
    d1, d2, d3, d4, d5 = (W1.shape[0], W2.shape[0], W3.shape[0],
                          W4.shape[0], W5.shape[0])
    p1, p2, p3, p4, p5 = _P
    W1p, W2p, W3p, W4p, W5p = _apply_masks_padded(
        W1, M1, W2, M2, W3, M3, W4, M4, W5, M5)
    b1p = jnp.pad(b1, (0, p1 - d1)).reshape(1, -1)
    b2p = jnp.pad(b2, (0, p2 - d2)).reshape(1, -1)
    b3p = jnp.pad(b3, (0, p3 - d3)).reshape(1, -1)
    b4p = jnp.pad(b4, (0, p4 - d4)).reshape(1, -1)
    b5p = jnp.pad(b5, (0, p5 - d5)).reshape(1, -1)

    def wspec(a):
        return pl.BlockSpec(a.shape, lambda i: (0, 0))

    block = _BLOCK
    h1, h2, h3, h4, h5 = pl.pallas_call(
        _mlp_kernel,
        grid=(n // block,),
        in_specs=[
            pl.BlockSpec((block, d_in), lambda i: (i, 0)),
            wspec(W1p), wspec(b1p),
            wspec(W2p), wspec(b2p),
            wspec(W3p), wspec(b3p),
            wspec(W4p), wspec(b4p),
            wspec(W5p), wspec(b5p),
        ],
        out_specs=[
            pl.BlockSpec((block, d1), lambda i: (i, 0)),
            pl.BlockSpec((block, d2), lambda i: (i, 0)),
            pl.BlockSpec((block, d3), lambda i: (i, 0)),
            pl.BlockSpec((block, d4), lambda i: (i, 0)),
            pl.BlockSpec((block, d5), lambda i: (i, 0)),
        ],
        out_shape=[
            jax.ShapeDtypeStruct((n, d1), jnp.float32),
            jax.ShapeDtypeStruct((n, d2), jnp.float32),
            jax.ShapeDtypeStruct((n, d3), jnp.float32),
            jax.ShapeDtypeStruct((n, d4), jnp.float32),
            jax.ShapeDtypeStruct((n, d5), jnp.float32),
        ],
        compiler_params=pltpu.CompilerParams(
            dimension_semantics=("parallel",),
        ),
    )(x, W1p, b1p, W2p, b2p, W3p, b3p, W4p, b4p, W5p, b5p)
    return (h5, h1, h2, h3, h4, h5)


# speedup vs baseline: 1.0253x; 1.0029x over previous
import jax
import jax.numpy as jnp
from jax.experimental import pallas as pl
from jax.experimental.pallas import tpu as pltpu

_BLOCK = 512
_P = (1024, 896, 512, 640, 128)


def _mask_pad_kernel(w1, m1, w2, m2, w3, m3, w4, m4, w5, m5,
                     o1, o2, o3, o4, o5):
    for w, m, o in ((w1, m1, o1), (w2, m2, o2), (w3, m3, o3),
                    (w4, m4, o4), (w5, m5, o5)):
        r, c = w.shape
        o[...] = jnp.zeros(o.shape, jnp.float32)
        o[0:r, 0:c] = w[...] * m[...]


def _apply_masks_padded(W1, M1, W2, M2, W3, M3, W4, M4, W5, M5):
    p1, p2, p3, p4, p5 = _P
    d_in = W1.shape[1]
    shapes = [jax.ShapeDtypeStruct(s, jnp.float32)
              for s in ((p1, d_in), (p2, p1), (p3, p2), (p4, p3), (p5, p4))]
    return pl.pallas_call(_mask_pad_kernel, out_shape=shapes)(
        W1, M1, W2, M2, W3, M3, W4, M4, W5, M5)


def _mlp_kernel(x_ref, w1_ref, b1_ref, w2_ref, b2_ref, w3_ref, b3_ref,
                w4_ref, b4_ref, w5_ref, b5_ref,
                h1_ref, h2_ref, h3_ref, h4_ref, h5_ref):
    dn = (((1,), (1,)), ((), ()))
    bf = jnp.bfloat16
    d1, d2, d3, d4, d5 = (r.shape[1] for r in
                          (h1_ref, h2_ref, h3_ref, h4_ref, h5_ref))

    x = x_ref[...].astype(bf)
    h1 = jax.lax.dot_general(x, w1_ref[...].astype(bf), dn,
                             preferred_element_type=jnp.float32)
    h1 = jnp.maximum(h1 + b1_ref[...], 0.0)
    h1_ref[...] = h1[:, :d1]

    h2 = jax.lax.dot_general(h1.astype(bf), w2_ref[...].astype(bf), dn,
                             preferred_element_type=jnp.float32)
    h2 = jnp.maximum(h2 + b2_ref[...], 0.0)
    h2_ref[...] = h2[:, :d2]

    h3 = jax.lax.dot_general(h2.astype(bf), w3_ref[...].astype(bf), dn,
                             preferred_element_type=jnp.float32)
    h3 = jnp.maximum(h3 + b3_ref[...], 0.0)
    h3_ref[...] = h3[:, :d3]

    h4 = jax.lax.dot_general(h3.astype(bf), w4_ref[...].astype(bf), dn,
                             preferred_element_type=jnp.float32)
    h4 = h4 + b4_ref[...]
    h4_ref[...] = h4[:, :d4]

    h5 = jax.lax.dot_general(h4.astype(bf), w5_ref[...].astype(bf), dn,
                             preferred_element_type=jnp.float32)
    h5 = h5 + b5_ref[...]
    h5_ref[...] = h5[:, :d5]


def kernel(x, W1, b1, M1, W2, b2, M2, W3, b3, M3, W4, b4, M4, W5, b5, M5):
    n, d_in = x.shape
    d1, d2, d3, d4, d5 = (W1.shape[0], W2.shape[0], W3.shape[0],
                          W4.shape[0], W5.shape[0])
    p1, p2, p3, p4, p5 = _P
    def _pad_to(a, rows, cols):
        return jnp.pad(a, ((0, rows - a.shape[0]), (0, cols - a.shape[1])))

    W1p = _pad_to(W1 * M1, p1, d_in)
    W2p = _pad_to(W2 * M2, p2, p1)
    W3p = _pad_to(W3 * M3, p3, p2)
    W4p = _pad_to(W4 * M4, p4, p3)
    W5p = _pad_to(W5 * M5, p5, p4)
    b1p = jnp.pad(b1, (0, p1 - d1)).reshape(1, -1)
    b2p = jnp.pad(b2, (0, p2 - d2)).reshape(1, -1)
    b3p = jnp.pad(b3, (0, p3 - d3)).reshape(1, -1)
    b4p = jnp.pad(b4, (0, p4 - d4)).reshape(1, -1)
    b5p = jnp.pad(b5, (0, p5 - d5)).reshape(1, -1)

    def wspec(a):
        return pl.BlockSpec(a.shape, lambda i: (0, 0))

    block = _BLOCK
    h1, h2, h3, h4, h5 = pl.pallas_call(
        _mlp_kernel,
        grid=(n // block,),
        in_specs=[
            pl.BlockSpec((block, d_in), lambda i: (i, 0)),
            wspec(W1p), wspec(b1p),
            wspec(W2p), wspec(b2p),
            wspec(W3p), wspec(b3p),
            wspec(W4p), wspec(b4p),
            wspec(W5p), wspec(b5p),
        ],
        out_specs=[
            pl.BlockSpec((block, d1), lambda i: (i, 0)),
            pl.BlockSpec((block, d2), lambda i: (i, 0)),
            pl.BlockSpec((block, d3), lambda i: (i, 0)),
            pl.BlockSpec((block, d4), lambda i: (i, 0)),
            pl.BlockSpec((block, d5), lambda i: (i, 0)),
        ],
        out_shape=[
            jax.ShapeDtypeStruct((n, d1), jnp.float32),
            jax.ShapeDtypeStruct((n, d2), jnp.float32),
            jax.ShapeDtypeStruct((n, d3), jnp.float32),
            jax.ShapeDtypeStruct((n, d4), jnp.float32),
            jax.ShapeDtypeStruct((n, d5), jnp.float32),
        ],
        compiler_params=pltpu.CompilerParams(
            dimension_semantics=("parallel",),
        ),
    )(x, W1p, b1p, W2p, b2p, W3p, b3p, W4p, b4p, W5p, b5p)
    return (h5, h1, h2, h3, h4, h5)
